# baseline (speedup 1.0000x reference)
import functools
import math

import jax
import jax.numpy as jnp
from jax.experimental import pallas as pl
from jax.experimental.pallas import tpu as pltpu
from jax.experimental.pallas import tpu_sc as plsc

_NC = 2
_NS = 16
_NW = _NC * _NS
_CH = 32


def _routing_body(cap, trash, nblk, x_ref, wg_ref, bg_ref,
                  gi_ref, lbl_ref, lg_ref):
    i = pl.program_id(0)
    blk = x_ref.shape[0]
    lg_ref[pl.ds(i * blk, blk), :] = (
        jnp.dot(x_ref[...], wg_ref[...], preferred_element_type=jnp.float32)
        + bg_ref[...]
    )

    @pl.when(i == nblk - 1)
    def _():
        _routing_tail(cap, trash, lg_ref, gi_ref, lbl_ref)


def _routing_tail(cap, trash, l_ref, gi_ref, lbl_ref):
    n, e = l_ref.shape
    l = l_ref[...]
    iota_e = jax.lax.broadcasted_iota(jnp.int32, (n, e), 1)
    rowmax = jnp.max(l, axis=1, keepdims=True)
    assign = jnp.min(jnp.where(l >= rowmax, iota_e, e), axis=1, keepdims=True)
    m = (iota_e == assign).astype(jnp.float32)

    ngrp = 8
    gs = n // ngrp
    gidx = jax.lax.broadcasted_iota(jnp.int32, (ngrp, n), 0)
    tidx = jax.lax.broadcasted_iota(jnp.int32, (ngrp, n), 1)
    sel = (tidx < gidx * gs).astype(jnp.float32)
    off = jnp.dot(sel, m, preferred_element_type=jnp.float32)
    rr = jax.lax.broadcasted_iota(jnp.int32, (gs, gs), 0)
    cc = jax.lax.broadcasted_iota(jnp.int32, (gs, gs), 1)
    ltri = (cc <= rr).astype(jnp.float32)
    parts = []
    for g in range(ngrp):
        w = jnp.dot(ltri, m[g * gs:(g + 1) * gs, :],
                    preferred_element_type=jnp.float32)
        parts.append(w + off[g:g + 1, :])
    pos = jnp.concatenate(parts, axis=0)

    pos_a = jnp.sum(m * pos, axis=1, keepdims=True)
    kept = pos_a <= cap
    slot = pos_a.astype(jnp.int32) - 1
    gi_ref[...] = jnp.where(kept, assign * cap + slot, trash)

    counts = jnp.sum(m, axis=0, keepdims=True)
    mean = jnp.sum(counts) / e
    var = jnp.sum((counts - mean) ** 2) / (e - 1)
    lbl_ref[...] = jnp.broadcast_to(jnp.sqrt(var) / mean, (1, 1))


def _router(xf, Wg, bg, cap, trash):
    n, dim = xf.shape
    e = Wg.shape[1]
    blk = 1024
    nblk = n // blk
    return pl.pallas_call(
        functools.partial(_routing_body, cap, trash, nblk),
        grid=(nblk,),
        in_specs=[
            pl.BlockSpec((blk, dim), lambda i: (i, 0)),
            pl.BlockSpec((dim, e), lambda i: (0, 0)),
            pl.BlockSpec((1, e), lambda i: (0, 0)),
        ],
        out_specs=[
            pl.BlockSpec((n, 1), lambda i: (0, 0)),
            pl.BlockSpec((1, 1), lambda i: (0, 0)),
        ],
        out_shape=[
            jax.ShapeDtypeStruct((n, 1), jnp.int32),
            jax.ShapeDtypeStruct((1, 1), jnp.float32),
        ],
        scratch_shapes=[pltpu.VMEM((n, e), jnp.float32)],
    )(xf, Wg, bg.reshape(1, e))


def _dispatch(xf, gi, rows_total, trash):
    n, dim = xf.shape
    per_w = n // _NW
    nch = per_w // _CH
    mesh = plsc.VectorSubcoreMesh(core_axis_name="c", subcore_axis_name="s")

    @functools.partial(
        pl.kernel,
        out_type=[
            jax.ShapeDtypeStruct((rows_total, dim), jnp.float32),
            jax.ShapeDtypeStruct((rows_total, dim), jnp.float32),
        ],
        mesh=mesh,
        scratch_types=[
            pltpu.VMEM((nch, _CH), jnp.int32),
            pltpu.VMEM((2, _CH, dim), jnp.float32),
            pltpu.VMEM((8, dim), jnp.float32),
            pltpu.SemaphoreType.DMA((2,)),
            pltpu.SemaphoreType.DMA((2,)),
        ],
    )
    def k(x_hbm, i_hbm, xg_hbm, yg0_hbm, idx_v, rows_v, zrow_v, lsem, ssem):
        wid = jax.lax.axis_index("s") * _NC + jax.lax.axis_index("c")
        base = wid * per_w
        pltpu.sync_copy(i_hbm.at[wid], idx_v)

        @pl.when(wid == 0)
        def _():
            for r in range(8):
                @pl.loop(0, dim // 16)
                def _(j):
                    zrow_v[r, pl.ds(j * 16, 16)] = jnp.zeros((16,),
                                                             jnp.float32)
            pltpu.sync_copy(zrow_v, yg0_hbm.at[pl.ds(trash, 8)])
        loads = [
            pltpu.make_async_copy(
                x_hbm.at[pl.ds(base + ci * _CH, _CH)],
                rows_v.at[ci % 2], lsem.at[ci % 2])
            for ci in range(nch)
        ]
        scats = [
            pltpu.make_async_copy(
                rows_v.at[ci % 2], xg_hbm.at[idx_v.at[ci]], ssem.at[ci % 2])
            for ci in range(nch)
        ]
        loads[0].start()
        for ci in range(nch):
            loads[ci].wait()
            scats[ci].start()
            if ci + 1 < nch:
                if ci >= 1:
                    scats[ci - 1].wait()
                loads[ci + 1].start()
        scats[nch - 2].wait()
        scats[nch - 1].wait()

    return k(xf, gi)


def _combine(yg, gi, n):
    dim = yg.shape[1]
    per_w = n // _NW
    nch = per_w // _CH
    mesh = plsc.VectorSubcoreMesh(core_axis_name="c", subcore_axis_name="s")

    @functools.partial(
        pl.kernel,
        out_type=jax.ShapeDtypeStruct((n, dim), jnp.float32),
        mesh=mesh,
        scratch_types=[
            pltpu.VMEM((nch, _CH), jnp.int32),
            pltpu.VMEM((2, _CH, dim), jnp.float32),
            pltpu.SemaphoreType.DMA((2,)),
            pltpu.SemaphoreType.DMA((2,)),
        ],
    )
    def k(yg_hbm, i_hbm, o_hbm, idx_v, rows_v, gsem, wsem):
        wid = jax.lax.axis_index("s") * _NC + jax.lax.axis_index("c")
        base = wid * per_w
        pltpu.sync_copy(i_hbm.at[wid], idx_v)
        gaths = [
            pltpu.make_async_copy(
                yg_hbm.at[idx_v.at[ci]], rows_v.at[ci % 2], gsem.at[ci % 2])
            for ci in range(nch)
        ]
        writes = [
            pltpu.make_async_copy(
                rows_v.at[ci % 2],
                o_hbm.at[pl.ds(base + ci * _CH, _CH)], wsem.at[ci % 2])
            for ci in range(nch)
        ]
        gaths[0].start()
        for ci in range(nch):
            gaths[ci].wait()
            writes[ci].start()
            if ci + 1 < nch:
                if ci >= 1:
                    writes[ci - 1].wait()
                gaths[ci + 1].start()
        writes[nch - 2].wait()
        writes[nch - 1].wait()

    return k(yg, gi)


def _mlp_body(x_ref, w1_ref, b1_ref, w2_ref, b2_ref, yg0_ref, o_ref):
    h_id = pl.program_id(1)
    h = (
        jnp.dot(x_ref[...], w1_ref[0], preferred_element_type=jnp.float32)
        + b1_ref[0]
    )
    h = 0.5 * h * (1.0 + jax.lax.erf(h * (1.0 / math.sqrt(2.0))))
    part = jnp.dot(h, w2_ref[0], preferred_element_type=jnp.float32)

    @pl.when(h_id == 0)
    def _():
        o_ref[...] = part + b2_ref[0]

    @pl.when(h_id != 0)
    def _():
        o_ref[...] += part


def _expert_mlp(xg, yg0, W1, b1, W2, b2, cap, rows_total):
    e, dim, hid = W1.shape
    nh = 8
    ht = hid // nh
    return pl.pallas_call(
        _mlp_body,
        grid=(e, nh),
        in_specs=[
            pl.BlockSpec((cap, dim), lambda i, h: (i, 0)),
            pl.BlockSpec((1, dim, ht), lambda i, h: (i, 0, h)),
            pl.BlockSpec((1, 1, ht), lambda i, h: (i, 0, h)),
            pl.BlockSpec((1, ht, dim), lambda i, h: (i, h, 0)),
            pl.BlockSpec((1, 1, dim), lambda i, h: (i, 0, 0)),
            pl.BlockSpec(memory_space=pl.ANY),
        ],
        out_specs=pl.BlockSpec((cap, dim), lambda i, h: (i, 0)),
        out_shape=jax.ShapeDtypeStruct((rows_total, dim), jnp.float32),
        input_output_aliases={5: 0},
        compiler_params=pltpu.CompilerParams(
            dimension_semantics=("parallel", "arbitrary"),
        ),
    )(xg, W1, b1.reshape(e, 1, hid), W2, b2.reshape(e, 1, dim), yg0)


def kernel(x, Wg, bg, W1, b1, W2, b2):
    b, s, dim = x.shape
    e = Wg.shape[1]
    n = b * s
    cap = int(1.25 * s * b / e)
    trash = e * cap
    rows_total = e * cap + 128

    xf = x.reshape(n, dim)
    gi, lbl = _router(xf, Wg, bg, cap, trash)
    nch = n // _NW // _CH
    gi3 = gi.reshape(_NW, nch, _CH)
    xg, yg0 = _dispatch(xf, gi3, rows_total, trash)
    yg = _expert_mlp(xg, yg0, W1, b1, W2, b2, cap, rows_total)
    out = _combine(yg, gi3, n)
    return out.reshape(b, s, dim), lbl[0, 0]


# speedup vs baseline: 1.1080x; 1.1080x over previous
import functools
import math

import jax
import jax.numpy as jnp
from jax.experimental import pallas as pl
from jax.experimental.pallas import tpu as pltpu
from jax.experimental.pallas import tpu_sc as plsc

_NC = 2
_NS = 16
_NW = _NC * _NS
_CH = 32


def _routing_body(cap, trash, nblk, x_ref, wg_ref, bg_ref,
                  gi_ref, lbl_ref, carry_ref):
    i = pl.program_id(0)
    blk, e = x_ref.shape[0], wg_ref.shape[1]

    @pl.when(i == 0)
    def _():
        carry_ref[...] = jnp.zeros((1, e), jnp.float32)

    l = (
        jnp.dot(x_ref[...], wg_ref[...], preferred_element_type=jnp.float32)
        + bg_ref[...]
    )
    iota_e = jax.lax.broadcasted_iota(jnp.int32, (blk, e), 1)
    rowmax = jnp.max(l, axis=1, keepdims=True)
    assign = jnp.min(jnp.where(l >= rowmax, iota_e, e), axis=1, keepdims=True)
    m = (iota_e == assign).astype(jnp.float32)

    rr = jax.lax.broadcasted_iota(jnp.int32, (blk, blk), 0)
    cc = jax.lax.broadcasted_iota(jnp.int32, (blk, blk), 1)
    ltri = (cc <= rr).astype(jnp.float32)
    within = jnp.dot(ltri, m, preferred_element_type=jnp.float32)
    pos = within + carry_ref[...]
    carry_ref[...] += within[blk - 1:blk, :]

    pos_a = jnp.sum(m * pos, axis=1, keepdims=True)
    kept = pos_a <= cap
    slot = pos_a.astype(jnp.int32) - 1
    gi_ref[...] = jnp.where(kept, assign * cap + slot, trash)

    @pl.when(i == nblk - 1)
    def _():
        counts = carry_ref[...]
        mean = jnp.sum(counts) / e
        var = jnp.sum((counts - mean) ** 2) / (e - 1)
        lbl_ref[...] = jnp.broadcast_to(jnp.sqrt(var) / mean, (1, 1))


def _router(xf, Wg, bg, cap, trash):
    n, dim = xf.shape
    e = Wg.shape[1]
    blk = 1024
    nblk = n // blk
    return pl.pallas_call(
        functools.partial(_routing_body, cap, trash, nblk),
        grid=(nblk,),
        in_specs=[
            pl.BlockSpec((blk, dim), lambda i: (i, 0)),
            pl.BlockSpec((dim, e), lambda i: (0, 0)),
            pl.BlockSpec((1, e), lambda i: (0, 0)),
        ],
        out_specs=[
            pl.BlockSpec((blk, 1), lambda i: (i, 0)),
            pl.BlockSpec((1, 1), lambda i: (0, 0)),
        ],
        out_shape=[
            jax.ShapeDtypeStruct((n, 1), jnp.int32),
            jax.ShapeDtypeStruct((1, 1), jnp.float32),
        ],
        scratch_shapes=[pltpu.VMEM((1, e), jnp.float32)],
    )(xf, Wg, bg.reshape(1, e))


def _dispatch(xf, gi, rows_total, trash):
    n, dim = xf.shape
    per_w = n // _NW
    nch = per_w // _CH
    mesh = plsc.VectorSubcoreMesh(core_axis_name="c", subcore_axis_name="s")

    @functools.partial(
        pl.kernel,
        out_type=[
            jax.ShapeDtypeStruct((rows_total, dim), jnp.float32),
            jax.ShapeDtypeStruct((rows_total, dim), jnp.float32),
        ],
        mesh=mesh,
        scratch_types=[
            pltpu.VMEM((nch, _CH), jnp.int32),
            pltpu.VMEM((2, _CH, dim), jnp.float32),
            pltpu.VMEM((8, dim), jnp.float32),
            pltpu.SemaphoreType.DMA((2,)),
            pltpu.SemaphoreType.DMA((2,)),
        ],
    )
    def k(x_hbm, i_hbm, xg_hbm, yg0_hbm, idx_v, rows_v, zrow_v, lsem, ssem):
        wid = jax.lax.axis_index("s") * _NC + jax.lax.axis_index("c")
        base = wid * per_w
        pltpu.sync_copy(i_hbm.at[wid], idx_v)

        @pl.when(wid == 0)
        def _():
            for r in range(8):
                @pl.loop(0, dim // 16)
                def _(j):
                    zrow_v[r, pl.ds(j * 16, 16)] = jnp.zeros((16,),
                                                             jnp.float32)
            pltpu.sync_copy(zrow_v, yg0_hbm.at[pl.ds(trash, 8)])
        loads = [
            pltpu.make_async_copy(
                x_hbm.at[pl.ds(base + ci * _CH, _CH)],
                rows_v.at[ci % 2], lsem.at[ci % 2])
            for ci in range(nch)
        ]
        scats = [
            pltpu.make_async_copy(
                rows_v.at[ci % 2], xg_hbm.at[idx_v.at[ci]], ssem.at[ci % 2])
            for ci in range(nch)
        ]
        loads[0].start()
        for ci in range(nch):
            loads[ci].wait()
            scats[ci].start()
            if ci + 1 < nch:
                if ci >= 1:
                    scats[ci - 1].wait()
                loads[ci + 1].start()
        scats[nch - 2].wait()
        scats[nch - 1].wait()

    return k(xf, gi)


def _combine(yg, gi, n):
    dim = yg.shape[1]
    per_w = n // _NW
    nch = per_w // _CH
    mesh = plsc.VectorSubcoreMesh(core_axis_name="c", subcore_axis_name="s")

    @functools.partial(
        pl.kernel,
        out_type=jax.ShapeDtypeStruct((n, dim), jnp.float32),
        mesh=mesh,
        scratch_types=[
            pltpu.VMEM((nch, _CH), jnp.int32),
            pltpu.VMEM((2, _CH, dim), jnp.float32),
            pltpu.SemaphoreType.DMA((2,)),
            pltpu.SemaphoreType.DMA((2,)),
        ],
    )
    def k(yg_hbm, i_hbm, o_hbm, idx_v, rows_v, gsem, wsem):
        wid = jax.lax.axis_index("s") * _NC + jax.lax.axis_index("c")
        base = wid * per_w
        pltpu.sync_copy(i_hbm.at[wid], idx_v)
        gaths = [
            pltpu.make_async_copy(
                yg_hbm.at[idx_v.at[ci]], rows_v.at[ci % 2], gsem.at[ci % 2])
            for ci in range(nch)
        ]
        writes = [
            pltpu.make_async_copy(
                rows_v.at[ci % 2],
                o_hbm.at[pl.ds(base + ci * _CH, _CH)], wsem.at[ci % 2])
            for ci in range(nch)
        ]
        gaths[0].start()
        for ci in range(nch):
            gaths[ci].wait()
            writes[ci].start()
            if ci + 1 < nch:
                if ci >= 1:
                    writes[ci - 1].wait()
                gaths[ci + 1].start()
        writes[nch - 2].wait()
        writes[nch - 1].wait()

    return k(yg, gi)


def _mlp_body(x_ref, w1_ref, b1_ref, w2_ref, b2_ref, yg0_ref, o_ref):
    h_id = pl.program_id(1)
    h = (
        jnp.dot(x_ref[...], w1_ref[0], preferred_element_type=jnp.float32)
        + b1_ref[0]
    )
    h = 0.5 * h * (1.0 + jax.lax.erf(h * (1.0 / math.sqrt(2.0))))
    part = jnp.dot(h, w2_ref[0], preferred_element_type=jnp.float32)

    @pl.when(h_id == 0)
    def _():
        o_ref[...] = part + b2_ref[0]

    @pl.when(h_id != 0)
    def _():
        o_ref[...] += part


def _expert_mlp(xg, yg0, W1, b1, W2, b2, cap, rows_total):
    e, dim, hid = W1.shape
    nh = 4
    ht = hid // nh
    return pl.pallas_call(
        _mlp_body,
        grid=(e, nh),
        in_specs=[
            pl.BlockSpec((cap, dim), lambda i, h: (i, 0)),
            pl.BlockSpec((1, dim, ht), lambda i, h: (i, 0, h)),
            pl.BlockSpec((1, 1, ht), lambda i, h: (i, 0, h)),
            pl.BlockSpec((1, ht, dim), lambda i, h: (i, h, 0)),
            pl.BlockSpec((1, 1, dim), lambda i, h: (i, 0, 0)),
            pl.BlockSpec(memory_space=pl.ANY),
        ],
        out_specs=pl.BlockSpec((cap, dim), lambda i, h: (i, 0)),
        out_shape=jax.ShapeDtypeStruct((rows_total, dim), jnp.float32),
        input_output_aliases={5: 0},
        compiler_params=pltpu.CompilerParams(
            dimension_semantics=("parallel", "arbitrary"),
        ),
    )(xg, W1, b1.reshape(e, 1, hid), W2, b2.reshape(e, 1, dim), yg0)


def kernel(x, Wg, bg, W1, b1, W2, b2):
    b, s, dim = x.shape
    e = Wg.shape[1]
    n = b * s
    cap = int(1.25 * s * b / e)
    trash = e * cap
    rows_total = e * cap + 128

    xf = x.reshape(n, dim)
    gi, lbl = _router(xf, Wg, bg, cap, trash)
    nch = n // _NW // _CH
    gi3 = gi.reshape(_NW, nch, _CH)
    xg, yg0 = _dispatch(xf, gi3, rows_total, trash)
    yg = _expert_mlp(xg, yg0, W1, b1, W2, b2, cap, rows_total)
    out = _combine(yg, gi3, n)
    return out.reshape(b, s, dim), lbl[0, 0]


# speedup vs baseline: 1.1190x; 1.0099x over previous
import functools
import math

import jax
import jax.numpy as jnp
from jax.experimental import pallas as pl
from jax.experimental.pallas import tpu as pltpu
from jax.experimental.pallas import tpu_sc as plsc

_NC = 2
_NS = 16
_NW = _NC * _NS
_CH = 64


def _routing_body(cap, trash, nblk, x_ref, wg_ref, bg_ref,
                  gi_ref, lbl_ref, carry_ref):
    i = pl.program_id(0)
    blk, e = x_ref.shape[0], wg_ref.shape[1]

    @pl.when(i == 0)
    def _():
        carry_ref[...] = jnp.zeros((1, e), jnp.float32)

    l = (
        jnp.dot(x_ref[...], wg_ref[...], preferred_element_type=jnp.float32)
        + bg_ref[...]
    )
    iota_e = jax.lax.broadcasted_iota(jnp.int32, (blk, e), 1)
    rowmax = jnp.max(l, axis=1, keepdims=True)
    assign = jnp.min(jnp.where(l >= rowmax, iota_e, e), axis=1, keepdims=True)
    m = (iota_e == assign).astype(jnp.float32)

    rr = jax.lax.broadcasted_iota(jnp.int32, (blk, blk), 0)
    cc = jax.lax.broadcasted_iota(jnp.int32, (blk, blk), 1)
    ltri = (cc <= rr).astype(jnp.float32)
    within = jnp.dot(ltri, m, preferred_element_type=jnp.float32)
    pos = within + carry_ref[...]
    carry_ref[...] += within[blk - 1:blk, :]

    pos_a = jnp.sum(m * pos, axis=1, keepdims=True)
    kept = pos_a <= cap
    slot = pos_a.astype(jnp.int32) - 1
    gi_ref[...] = jnp.where(kept, assign * cap + slot, trash)

    @pl.when(i == nblk - 1)
    def _():
        counts = carry_ref[...]
        mean = jnp.sum(counts) / e
        var = jnp.sum((counts - mean) ** 2) / (e - 1)
        lbl_ref[...] = jnp.broadcast_to(jnp.sqrt(var) / mean, (1, 1))


def _router(xf, Wg, bg, cap, trash):
    n, dim = xf.shape
    e = Wg.shape[1]
    blk = 1024
    nblk = n // blk
    return pl.pallas_call(
        functools.partial(_routing_body, cap, trash, nblk),
        grid=(nblk,),
        in_specs=[
            pl.BlockSpec((blk, dim), lambda i: (i, 0)),
            pl.BlockSpec((dim, e), lambda i: (0, 0)),
            pl.BlockSpec((1, e), lambda i: (0, 0)),
        ],
        out_specs=[
            pl.BlockSpec((blk, 1), lambda i: (i, 0)),
            pl.BlockSpec((1, 1), lambda i: (0, 0)),
        ],
        out_shape=[
            jax.ShapeDtypeStruct((n, 1), jnp.int32),
            jax.ShapeDtypeStruct((1, 1), jnp.float32),
        ],
        scratch_shapes=[pltpu.VMEM((1, e), jnp.float32)],
    )(xf, Wg, bg.reshape(1, e))


def _dispatch(xf, gi, rows_total, trash):
    n, dim = xf.shape
    per_w = n // _NW
    nch = per_w // _CH
    mesh = plsc.VectorSubcoreMesh(core_axis_name="c", subcore_axis_name="s")

    @functools.partial(
        pl.kernel,
        out_type=[
            jax.ShapeDtypeStruct((rows_total, dim), jnp.float32),
            jax.ShapeDtypeStruct((rows_total, dim), jnp.float32),
        ],
        mesh=mesh,
        scratch_types=[
            pltpu.VMEM((nch, _CH), jnp.int32),
            pltpu.VMEM((1, _CH, dim), jnp.float32),
            pltpu.VMEM((8, dim), jnp.float32),
        ],
    )
    def k(x_hbm, i_hbm, xg_hbm, yg0_hbm, idx_v, rows_v, zrow_v):
        wid = jax.lax.axis_index("s") * _NC + jax.lax.axis_index("c")
        base = wid * per_w
        pltpu.sync_copy(i_hbm.at[wid], idx_v)

        @pl.when(wid == 0)
        def _():
            for r in range(8):
                @pl.loop(0, dim // 16)
                def _(j):
                    zrow_v[r, pl.ds(j * 16, 16)] = jnp.zeros((16,),
                                                             jnp.float32)
            pltpu.sync_copy(zrow_v, yg0_hbm.at[pl.ds(trash, 8)])
        for ci in range(nch):
            pltpu.sync_copy(x_hbm.at[pl.ds(base + ci * _CH, _CH)],
                            rows_v.at[0])
            pltpu.sync_copy(rows_v.at[0], xg_hbm.at[idx_v.at[ci]])

    return k(xf, gi)


def _combine(yg, gi, n):
    dim = yg.shape[1]
    per_w = n // _NW
    nch = per_w // _CH
    mesh = plsc.VectorSubcoreMesh(core_axis_name="c", subcore_axis_name="s")

    @functools.partial(
        pl.kernel,
        out_type=jax.ShapeDtypeStruct((n, dim), jnp.float32),
        mesh=mesh,
        scratch_types=[
            pltpu.VMEM((nch, _CH), jnp.int32),
            pltpu.VMEM((1, _CH, dim), jnp.float32),
        ],
    )
    def k(yg_hbm, i_hbm, o_hbm, idx_v, rows_v):
        wid = jax.lax.axis_index("s") * _NC + jax.lax.axis_index("c")
        base = wid * per_w
        pltpu.sync_copy(i_hbm.at[wid], idx_v)
        for ci in range(nch):
            pltpu.sync_copy(yg_hbm.at[idx_v.at[ci]], rows_v.at[0])
            pltpu.sync_copy(rows_v.at[0],
                            o_hbm.at[pl.ds(base + ci * _CH, _CH)])

    return k(yg, gi)


def _mlp_body(x_ref, w1_ref, b1_ref, w2_ref, b2_ref, yg0_ref, o_ref):
    h_id = pl.program_id(1)
    h = (
        jnp.dot(x_ref[...], w1_ref[0], preferred_element_type=jnp.float32)
        + b1_ref[0]
    )
    h = 0.5 * h * (1.0 + jax.lax.erf(h * (1.0 / math.sqrt(2.0))))
    part = jnp.dot(h, w2_ref[0], preferred_element_type=jnp.float32)

    @pl.when(h_id == 0)
    def _():
        o_ref[...] = part + b2_ref[0]

    @pl.when(h_id != 0)
    def _():
        o_ref[...] += part


def _expert_mlp(xg, yg0, W1, b1, W2, b2, cap, rows_total):
    e, dim, hid = W1.shape
    nh = 4
    ht = hid // nh
    return pl.pallas_call(
        _mlp_body,
        grid=(e, nh),
        in_specs=[
            pl.BlockSpec((cap, dim), lambda i, h: (i, 0)),
            pl.BlockSpec((1, dim, ht), lambda i, h: (i, 0, h)),
            pl.BlockSpec((1, 1, ht), lambda i, h: (i, 0, h)),
            pl.BlockSpec((1, ht, dim), lambda i, h: (i, h, 0)),
            pl.BlockSpec((1, 1, dim), lambda i, h: (i, 0, 0)),
            pl.BlockSpec(memory_space=pl.ANY),
        ],
        out_specs=pl.BlockSpec((cap, dim), lambda i, h: (i, 0)),
        out_shape=jax.ShapeDtypeStruct((rows_total, dim), jnp.float32),
        input_output_aliases={5: 0},
        compiler_params=pltpu.CompilerParams(
            dimension_semantics=("parallel", "arbitrary"),
        ),
    )(xg, W1, b1.reshape(e, 1, hid), W2, b2.reshape(e, 1, dim), yg0)


def kernel(x, Wg, bg, W1, b1, W2, b2):
    b, s, dim = x.shape
    e = Wg.shape[1]
    n = b * s
    cap = int(1.25 * s * b / e)
    trash = e * cap
    rows_total = e * cap + 128

    xf = x.reshape(n, dim)
    gi, lbl = _router(xf, Wg, bg, cap, trash)
    nch = n // _NW // _CH
    gi3 = gi.reshape(_NW, nch, _CH)
    xg, yg0 = _dispatch(xf, gi3, rows_total, trash)
    yg = _expert_mlp(xg, yg0, W1, b1, W2, b2, cap, rows_total)
    out = _combine(yg, gi3, n)
    return out.reshape(b, s, dim), lbl[0, 0]
